# baseline (speedup 1.0000x reference)
import functools

import jax
import jax.numpy as jnp
from jax import lax
from jax.experimental import pallas as pl
from jax.experimental.pallas import tpu as pltpu
from jax.experimental.pallas import tpu_sc as plsc

N_NODES = 10000
N_EDGES = 320000
D = 128
NC = 2
NS = 16
NW = NC * NS
EPT = N_EDGES // NW
CHUNK = 80
NCHUNK = EPT // CHUNK
NBUF = 4
N_PAD = 10240
ROWS_PT = N_PAD // NS

_mesh = plsc.VectorSubcoreMesh(core_axis_name="c", subcore_axis_name="s")


def _edge_body(h_hbm, src_hbm, dst_hbm, zeros_hbm, out_hbm,
               is0, is1, is2, is3, id0, id1, id2, id3,
               rw0, rw1, rw2, rw3, acc, sem_g, sem_s, sem_is, sem_id):
    cid = lax.axis_index("c")
    sid = lax.axis_index("s")
    r0 = sid * ROWS_PT
    pltpu.sync_copy(zeros_hbm.at[pl.ds(r0, ROWS_PT)], acc.at[pl.ds(r0, ROWS_PT)])
    base = (cid * NS + sid) * EPT

    isb = [is0, is1, is2, is3]
    idb = [id0, id1, id2, id3]
    rows = [rw0, rw1, rw2, rw3]

    def load_idx_sync(hbm, buf, i):
        pltpu.sync_copy(hbm.at[pl.ds(base + i * CHUNK, CHUNK)], buf)

    def load_idx_async(hbm, buf, sem, i, b):
        pltpu.async_copy(hbm.at[pl.ds(base + i * CHUNK, CHUNK)], buf[b], sem.at[b])

    def wait_idx(hbm, buf, sem, b):
        pltpu.make_async_copy(hbm.at[pl.ds(base, CHUNK)], buf[b], sem.at[b]).wait()

    def issue_gather(i, b):
        pltpu.async_copy(h_hbm.at[isb[b]], rows[b], sem_g.at[b])

    def wait_gather(b):
        pltpu.make_async_copy(h_hbm.at[isb[b]], rows[b], sem_g.at[b]).wait()

    def issue_scatter(i, b):
        pltpu.async_copy(rows[b], acc.at[idb[b]], sem_s.at[b], add=True)

    def wait_scatter(b):
        pltpu.make_async_copy(rows[b], acc.at[idb[b]], sem_s.at[b]).wait()

    for b in range(NBUF):
        load_idx_sync(src_hbm, isb[b], b)
    load_idx_sync(dst_hbm, idb[0], 0)
    load_idx_sync(dst_hbm, idb[1], 1)
    plsc.subcore_barrier()

    issue_gather(0, 0)
    issue_gather(1, 1)
    load_idx_async(dst_hbm, idb, sem_id, 2, 2)
    issue_gather(2, 2)
    wait_gather(0)
    load_idx_async(src_hbm, isb, sem_is, 4, 0)
    issue_scatter(0, 0)
    load_idx_async(dst_hbm, idb, sem_id, 3, 3)
    issue_gather(3, 3)
    wait_gather(1)
    load_idx_async(src_hbm, isb, sem_is, 5, 1)
    issue_scatter(1, 1)

    def steady(i, carry):
        for m in range(NBUF):
            ci = i + m
            p = (2 + m) % NBUF
            wait_scatter(m)
            load_idx_async(dst_hbm, idb, sem_id, ci + 2, m)
            wait_idx(src_hbm, isb, sem_is, m)
            issue_gather(ci + 2, m)
            wait_gather(p)
            load_idx_async(src_hbm, isb, sem_is, ci + 4, p)
            wait_idx(dst_hbm, idb, sem_id, p)
            issue_scatter(ci, p)
        return carry

    lax.fori_loop(0, 30, lambda j, c: steady(4 * j + 2, c), 0)

    wait_scatter(0)
    load_idx_async(dst_hbm, idb, sem_id, 124, 0)
    wait_idx(src_hbm, isb, sem_is, 0)
    issue_gather(124, 0)
    wait_gather(2)
    wait_idx(dst_hbm, idb, sem_id, 2)
    issue_scatter(122, 2)
    wait_gather(3)
    wait_idx(dst_hbm, idb, sem_id, 3)
    issue_scatter(123, 3)
    wait_gather(0)
    wait_idx(dst_hbm, idb, sem_id, 0)
    issue_scatter(124, 0)
    wait_idx(src_hbm, isb, sem_is, 1)
    wait_scatter(1)
    wait_scatter(2)
    wait_scatter(3)
    wait_scatter(0)

    plsc.subcore_barrier()
    pltpu.sync_copy(acc.at[pl.ds(r0, ROWS_PT)],
                    out_hbm.at[pl.ds(cid * N_PAD + r0, ROWS_PT)])


_edge_pass = functools.partial(
    pl.kernel,
    mesh=_mesh,
    out_type=jax.ShapeDtypeStruct((NC * N_PAD, D), jnp.float32),
    scratch_types=(
        [pltpu.VMEM((CHUNK,), jnp.int32)] * 8
        + [pltpu.VMEM((CHUNK, D), jnp.float32)] * 4
        + [
            pltpu.VMEM_SHARED((N_PAD, D), jnp.float32),
            pltpu.SemaphoreType.DMA((NBUF,)),
            pltpu.SemaphoreType.DMA((NBUF,)),
            pltpu.SemaphoreType.DMA((NBUF,)),
            pltpu.SemaphoreType.DMA((NBUF,)),
        ]
    ),
)(_edge_body)


def _deg_body(dst_hbm, ones_hbm, zeros_hbm, out_hbm,
              id0, id1, id2, id3, ones_v, acc, sem_s):
    cid = lax.axis_index("c")
    sid = lax.axis_index("s")
    r0 = sid * ROWS_PT
    pltpu.sync_copy(zeros_hbm.at[pl.ds(r0, ROWS_PT)], acc.at[pl.ds(r0, ROWS_PT)])
    pltpu.sync_copy(ones_hbm, ones_v)
    base = (cid * NS + sid) * EPT
    plsc.subcore_barrier()

    idb = [id0, id1, id2, id3]

    def issue_scatter(i, b):
        pltpu.sync_copy(dst_hbm.at[pl.ds(base + i * CHUNK, CHUNK)], idb[b])
        pltpu.async_copy(ones_v, acc.at[idb[b]], sem_s.at[b], add=True)

    def wait_scatter(b):
        pltpu.make_async_copy(ones_v, acc.at[idb[b]], sem_s.at[b]).wait()

    for b in range(NBUF):
        issue_scatter(b, b)

    def steady(i, carry):
        for m in range(NBUF):
            wait_scatter(m)
            issue_scatter(i + m, m)
        return carry

    lax.fori_loop(0, 30, lambda j, c: steady(4 * j + 4, c), 0)

    wait_scatter(0)
    issue_scatter(124, 0)
    wait_scatter(1)
    wait_scatter(2)
    wait_scatter(3)
    wait_scatter(0)

    plsc.subcore_barrier()
    pltpu.sync_copy(acc.at[pl.ds(r0, ROWS_PT)],
                    out_hbm.at[pl.ds(cid * N_PAD + r0, ROWS_PT)])


_deg_pass = functools.partial(
    pl.kernel,
    mesh=_mesh,
    out_type=jax.ShapeDtypeStruct((NC * N_PAD, D), jnp.float32),
    scratch_types=(
        [pltpu.VMEM((CHUNK,), jnp.int32)] * 4
        + [
            pltpu.VMEM((CHUNK, D), jnp.float32),
            pltpu.VMEM_SHARED((N_PAD, D), jnp.float32),
            pltpu.SemaphoreType.DMA((NBUF,)),
        ]
    ),
)(_deg_body)


def _dinv_of(deg_ref):
    deg = deg_ref[0:N_NODES] + deg_ref[N_PAD:N_PAD + N_NODES]
    degt = jnp.max(deg, axis=1, keepdims=True) + 1.0
    return lax.rsqrt(degt)


def _tca_body(x_ref, w_ref, o_ref):
    o_ref[...] = jnp.dot(x_ref[...], w_ref[...], preferred_element_type=jnp.float32)


def _tcb_body(h_ref, deg_ref, o_ref):
    o_ref[...] = h_ref[...] * _dinv_of(deg_ref)


def _tc2_body(agg_ref, hhat_ref, deg_ref, b_ref, w_ref, o_ref):
    dinv = _dinv_of(deg_ref)
    s = agg_ref[0:N_NODES] + agg_ref[N_PAD:N_PAD + N_NODES] + hhat_ref[...]
    a = jnp.maximum(s * dinv + b_ref[...], 0.0)
    h2 = jnp.dot(a, w_ref[...], preferred_element_type=jnp.float32)
    o_ref[...] = h2 * dinv


def _tc3_body(agg_ref, hhat_ref, deg_ref, b_ref, wr_ref, br_ref, o_ref):
    dinv = _dinv_of(deg_ref)
    s = agg_ref[0:N_NODES] + agg_ref[N_PAD:N_PAD + N_NODES] + hhat_ref[...]
    a = jnp.maximum(s * dinv + b_ref[...], 0.0)
    z = jnp.dot(a, wr_ref[...], preferred_element_type=jnp.float32) + br_ref[...]
    o_ref[...] = jax.nn.sigmoid(z) * 0.8 + 0.1


_f32 = jnp.float32
_tca = pl.pallas_call(
    _tca_body, out_shape=jax.ShapeDtypeStruct((N_NODES, D), _f32))
_tcb = pl.pallas_call(
    _tcb_body, out_shape=jax.ShapeDtypeStruct((N_NODES, D), _f32))
_tc2 = pl.pallas_call(
    _tc2_body, out_shape=jax.ShapeDtypeStruct((N_NODES, D), _f32))
_tc3 = pl.pallas_call(
    _tc3_body, out_shape=jax.ShapeDtypeStruct((N_NODES, D), _f32))


@jax.jit
def kernel(x, edge_index, W1, b1, W2, b2, Wr, br):
    src = edge_index[0].astype(jnp.int32)
    dst = edge_index[1].astype(jnp.int32)
    zeros128 = jnp.zeros((N_PAD, D), _f32)
    ones128 = jnp.ones((CHUNK, D), _f32)

    src_pad = jnp.concatenate([src, jnp.zeros((CHUNK,), jnp.int32)])

    h1 = _tca(x, W1)
    degp = _deg_pass(dst, ones128, zeros128)
    hhat = _tcb(h1, degp)
    agg1 = _edge_pass(hhat, src_pad, dst, zeros128)
    hhat2 = _tc2(agg1, hhat, degp, b1.reshape(1, D), W2)
    agg2 = _edge_pass(hhat2, src_pad, dst, zeros128)
    out = _tc3(agg2, hhat2, degp, b2.reshape(1, D), Wr, br.reshape(1, D))
    return out


# speedup vs baseline: 1.0062x; 1.0062x over previous
import functools

import jax
import jax.numpy as jnp
from jax import lax
from jax.experimental import pallas as pl
from jax.experimental.pallas import tpu as pltpu
from jax.experimental.pallas import tpu_sc as plsc

N_NODES = 10000
N_EDGES = 320000
D = 128
NC = 2
NS = 16
NW = NC * NS
EPT = N_EDGES // NW
CHUNK = 80
NCHUNK = EPT // CHUNK
NBUF = 4
N_PAD = 10240
ROWS_PT = N_PAD // NS

_mesh = plsc.VectorSubcoreMesh(core_axis_name="c", subcore_axis_name="s")


def _edge_body(h_hbm, src_hbm, dst_hbm, zeros_hbm, out_hbm,
               is0, is1, is2, is3, id0, id1, id2, id3,
               rw0, rw1, rw2, rw3, acc, sem_g, sem_s, sem_is, sem_id):
    cid = lax.axis_index("c")
    sid = lax.axis_index("s")
    r0 = sid * ROWS_PT
    pltpu.sync_copy(zeros_hbm.at[pl.ds(r0, ROWS_PT)], acc.at[pl.ds(r0, ROWS_PT)])
    base = (cid * NS + sid) * EPT

    isb = [is0, is1, is2, is3]
    idb = [id0, id1, id2, id3]
    rows = [rw0, rw1, rw2, rw3]

    def load_idx_sync(hbm, buf, i):
        pltpu.sync_copy(hbm.at[pl.ds(base + i * CHUNK, CHUNK)], buf)

    def load_idx_async(hbm, buf, sem, i, b):
        pltpu.async_copy(hbm.at[pl.ds(base + i * CHUNK, CHUNK)], buf[b], sem.at[b])

    def wait_idx(hbm, buf, sem, b):
        pltpu.make_async_copy(hbm.at[pl.ds(base, CHUNK)], buf[b], sem.at[b]).wait()

    def issue_gather(i, b):
        pltpu.async_copy(h_hbm.at[isb[b]], rows[b], sem_g.at[b])

    def wait_gather(b):
        pltpu.make_async_copy(h_hbm.at[isb[b]], rows[b], sem_g.at[b]).wait()

    def issue_scatter(i, b):
        pltpu.async_copy(rows[b], acc.at[idb[b]], sem_s.at[b], add=True)

    def wait_scatter(b):
        pltpu.make_async_copy(rows[b], acc.at[idb[b]], sem_s.at[b]).wait()

    for b in range(NBUF):
        load_idx_sync(src_hbm, isb[b], b)
    load_idx_sync(dst_hbm, idb[0], 0)
    load_idx_sync(dst_hbm, idb[1], 1)
    plsc.subcore_barrier()

    issue_gather(0, 0)
    issue_gather(1, 1)
    load_idx_async(dst_hbm, idb, sem_id, 2, 2)
    issue_gather(2, 2)
    wait_gather(0)
    load_idx_async(src_hbm, isb, sem_is, 4, 0)
    issue_scatter(0, 0)
    load_idx_async(dst_hbm, idb, sem_id, 3, 3)
    issue_gather(3, 3)
    wait_gather(1)
    load_idx_async(src_hbm, isb, sem_is, 5, 1)
    issue_scatter(1, 1)

    def steady(i, carry):
        for m in range(NBUF):
            ci = i + m
            p = (2 + m) % NBUF
            wait_scatter(m)
            load_idx_async(dst_hbm, idb, sem_id, ci + 2, m)
            wait_idx(src_hbm, isb, sem_is, m)
            issue_gather(ci + 2, m)
            wait_gather(p)
            load_idx_async(src_hbm, isb, sem_is, ci + 4, p)
            wait_idx(dst_hbm, idb, sem_id, p)
            issue_scatter(ci, p)
        return carry

    lax.fori_loop(0, 30, lambda j, c: steady(4 * j + 2, c), 0)

    wait_scatter(0)
    load_idx_async(dst_hbm, idb, sem_id, 124, 0)
    wait_idx(src_hbm, isb, sem_is, 0)
    issue_gather(124, 0)
    wait_gather(2)
    wait_idx(dst_hbm, idb, sem_id, 2)
    issue_scatter(122, 2)
    wait_gather(3)
    wait_idx(dst_hbm, idb, sem_id, 3)
    issue_scatter(123, 3)
    wait_gather(0)
    wait_idx(dst_hbm, idb, sem_id, 0)
    issue_scatter(124, 0)
    wait_idx(src_hbm, isb, sem_is, 1)
    wait_scatter(1)
    wait_scatter(2)
    wait_scatter(3)
    wait_scatter(0)

    plsc.subcore_barrier()
    pltpu.sync_copy(acc.at[pl.ds(r0, ROWS_PT)],
                    out_hbm.at[pl.ds(cid * N_PAD + r0, ROWS_PT)])


_edge_pass = functools.partial(
    pl.kernel,
    mesh=_mesh,
    out_type=jax.ShapeDtypeStruct((NC * N_PAD, D), jnp.float32),
    scratch_types=(
        [pltpu.VMEM((CHUNK,), jnp.int32)] * 8
        + [pltpu.VMEM((CHUNK, D), jnp.float32)] * 4
        + [
            pltpu.VMEM_SHARED((N_PAD, D), jnp.float32),
            pltpu.SemaphoreType.DMA((NBUF,)),
            pltpu.SemaphoreType.DMA((NBUF,)),
            pltpu.SemaphoreType.DMA((NBUF,)),
            pltpu.SemaphoreType.DMA((NBUF,)),
        ]
    ),
)(_edge_body)


def _deg_body(dst_hbm, ones_hbm, zeros_hbm, out_hbm,
              id0, id1, id2, id3, ones_v, acc, sem_s):
    cid = lax.axis_index("c")
    sid = lax.axis_index("s")
    r0 = sid * ROWS_PT
    pltpu.sync_copy(zeros_hbm.at[pl.ds(r0, ROWS_PT)], acc.at[pl.ds(r0, ROWS_PT)])
    pltpu.sync_copy(ones_hbm, ones_v)
    base = (cid * NS + sid) * EPT
    plsc.subcore_barrier()

    idb = [id0, id1, id2, id3]

    def issue_scatter(i, b):
        pltpu.sync_copy(dst_hbm.at[pl.ds(base + i * CHUNK, CHUNK)], idb[b])
        pltpu.async_copy(ones_v, acc.at[idb[b]], sem_s.at[b], add=True)

    def wait_scatter(b):
        pltpu.make_async_copy(ones_v, acc.at[idb[b]], sem_s.at[b]).wait()

    for b in range(NBUF):
        issue_scatter(b, b)

    def steady(i, carry):
        for m in range(NBUF):
            wait_scatter(m)
            issue_scatter(i + m, m)
        return carry

    lax.fori_loop(0, 30, lambda j, c: steady(4 * j + 4, c), 0)

    wait_scatter(0)
    issue_scatter(124, 0)
    wait_scatter(1)
    wait_scatter(2)
    wait_scatter(3)
    wait_scatter(0)

    plsc.subcore_barrier()
    pltpu.sync_copy(acc.at[pl.ds(r0, ROWS_PT)],
                    out_hbm.at[pl.ds(cid * N_PAD + r0, ROWS_PT)])


_deg_pass = functools.partial(
    pl.kernel,
    mesh=_mesh,
    out_type=jax.ShapeDtypeStruct((NC * N_PAD, D), jnp.float32),
    scratch_types=(
        [pltpu.VMEM((CHUNK,), jnp.int32)] * 4
        + [
            pltpu.VMEM((CHUNK, D), jnp.float32),
            pltpu.VMEM_SHARED((N_PAD, D), jnp.float32),
            pltpu.SemaphoreType.DMA((NBUF,)),
        ]
    ),
)(_deg_body)


def _dinv_of(deg_ref):
    deg = deg_ref[0:N_NODES] + deg_ref[N_PAD:N_PAD + N_NODES]
    degt = jnp.max(deg, axis=1, keepdims=True) + 1.0
    return lax.rsqrt(degt)


def _tc1_body(x_ref, w_ref, deg_ref, o_ref):
    dinv = _dinv_of(deg_ref)
    h = jnp.dot(x_ref[...], w_ref[...], preferred_element_type=jnp.float32)
    o_ref[...] = h * dinv


def _tc2_body(agg_ref, hhat_ref, deg_ref, b_ref, w_ref, o_ref):
    dinv = _dinv_of(deg_ref)
    s = agg_ref[0:N_NODES] + agg_ref[N_PAD:N_PAD + N_NODES] + hhat_ref[...]
    a = jnp.maximum(s * dinv + b_ref[...], 0.0)
    h2 = jnp.dot(a, w_ref[...], preferred_element_type=jnp.float32)
    o_ref[...] = h2 * dinv


def _tc3_body(agg_ref, hhat_ref, deg_ref, b_ref, wr_ref, br_ref, o_ref):
    dinv = _dinv_of(deg_ref)
    s = agg_ref[0:N_NODES] + agg_ref[N_PAD:N_PAD + N_NODES] + hhat_ref[...]
    a = jnp.maximum(s * dinv + b_ref[...], 0.0)
    z = jnp.dot(a, wr_ref[...], preferred_element_type=jnp.float32) + br_ref[...]
    o_ref[...] = jax.nn.sigmoid(z) * 0.8 + 0.1


_f32 = jnp.float32
_tc1 = pl.pallas_call(
    _tc1_body, out_shape=jax.ShapeDtypeStruct((N_NODES, D), _f32))
_tc2 = pl.pallas_call(
    _tc2_body, out_shape=jax.ShapeDtypeStruct((N_NODES, D), _f32))
_tc3 = pl.pallas_call(
    _tc3_body, out_shape=jax.ShapeDtypeStruct((N_NODES, D), _f32))


@jax.jit
def kernel(x, edge_index, W1, b1, W2, b2, Wr, br):
    src = edge_index[0].astype(jnp.int32)
    dst = edge_index[1].astype(jnp.int32)
    zeros128 = jnp.zeros((N_PAD, D), _f32)
    ones128 = jnp.ones((CHUNK, D), _f32)

    src_pad = jnp.concatenate([src, jnp.zeros((CHUNK,), jnp.int32)])

    degp = _deg_pass(dst, ones128, zeros128)
    hhat = _tc1(x, W1, degp)
    agg1 = _edge_pass(hhat, src_pad, dst, zeros128)
    hhat2 = _tc2(agg1, hhat, degp, b1.reshape(1, D), W2)
    agg2 = _edge_pass(hhat2, src_pad, dst, zeros128)
    out = _tc3(agg2, hhat2, degp, b2.reshape(1, D), Wr, br.reshape(1, D))
    return out


# speedup vs baseline: 1.0121x; 1.0059x over previous
import functools

import numpy as np

import jax
import jax.numpy as jnp
from jax import lax
from jax.experimental import pallas as pl
from jax.experimental.pallas import tpu as pltpu
from jax.experimental.pallas import tpu_sc as plsc

N_NODES = 10000
N_EDGES = 320000
D = 128
NC = 2
NS = 16
NW = NC * NS
EPT = N_EDGES // NW
CHUNK = 80
NCHUNK = EPT // CHUNK
NBUF = 4
N_PAD = 10240
ROWS_PT = N_PAD // NS

_mesh = plsc.VectorSubcoreMesh(core_axis_name="c", subcore_axis_name="s")

_ZEROS128 = np.zeros((N_PAD, D), np.float32)
_ONES128 = np.ones((CHUNK, D), np.float32)


def _edge_body(h_hbm, src_hbm, dst_hbm, zeros_hbm, out_hbm,
               is0, is1, is2, is3, id0, id1, id2, id3,
               rw0, rw1, rw2, rw3, acc, sem_g, sem_s, sem_is, sem_id):
    cid = lax.axis_index("c")
    sid = lax.axis_index("s")
    r0 = sid * ROWS_PT
    pltpu.sync_copy(zeros_hbm.at[pl.ds(r0, ROWS_PT)], acc.at[pl.ds(r0, ROWS_PT)])
    base = (cid * NS + sid) * EPT

    isb = [is0, is1, is2, is3]
    idb = [id0, id1, id2, id3]
    rows = [rw0, rw1, rw2, rw3]

    def load_idx_sync(hbm, buf, i):
        pltpu.sync_copy(hbm.at[pl.ds(base + i * CHUNK, CHUNK)], buf)

    def load_idx_async(hbm, buf, sem, i, b):
        pltpu.async_copy(hbm.at[pl.ds(base + i * CHUNK, CHUNK)], buf[b], sem.at[b])

    def wait_idx(hbm, buf, sem, b):
        pltpu.make_async_copy(hbm.at[pl.ds(base, CHUNK)], buf[b], sem.at[b]).wait()

    def issue_gather(i, b):
        pltpu.async_copy(h_hbm.at[isb[b]], rows[b], sem_g.at[b])

    def wait_gather(b):
        pltpu.make_async_copy(h_hbm.at[isb[b]], rows[b], sem_g.at[b]).wait()

    def issue_scatter(i, b):
        pltpu.async_copy(rows[b], acc.at[idb[b]], sem_s.at[b], add=True)

    def wait_scatter(b):
        pltpu.make_async_copy(rows[b], acc.at[idb[b]], sem_s.at[b]).wait()

    for b in range(NBUF):
        load_idx_sync(src_hbm, isb[b], b)
    load_idx_sync(dst_hbm, idb[0], 0)
    load_idx_sync(dst_hbm, idb[1], 1)
    plsc.subcore_barrier()

    issue_gather(0, 0)
    issue_gather(1, 1)
    load_idx_async(dst_hbm, idb, sem_id, 2, 2)
    issue_gather(2, 2)
    wait_gather(0)
    load_idx_async(src_hbm, isb, sem_is, 4, 0)
    issue_scatter(0, 0)
    load_idx_async(dst_hbm, idb, sem_id, 3, 3)
    issue_gather(3, 3)
    wait_gather(1)
    load_idx_async(src_hbm, isb, sem_is, 5, 1)
    issue_scatter(1, 1)

    def steady(i, carry):
        for m in range(NBUF):
            ci = i + m
            p = (2 + m) % NBUF
            wait_scatter(m)
            load_idx_async(dst_hbm, idb, sem_id, ci + 2, m)
            wait_idx(src_hbm, isb, sem_is, m)
            issue_gather(ci + 2, m)
            wait_gather(p)
            load_idx_async(src_hbm, isb, sem_is, ci + 4, p)
            wait_idx(dst_hbm, idb, sem_id, p)
            issue_scatter(ci, p)
        return carry

    lax.fori_loop(0, 30, lambda j, c: steady(4 * j + 2, c), 0)

    wait_scatter(0)
    load_idx_async(dst_hbm, idb, sem_id, 124, 0)
    wait_idx(src_hbm, isb, sem_is, 0)
    issue_gather(124, 0)
    wait_gather(2)
    wait_idx(dst_hbm, idb, sem_id, 2)
    issue_scatter(122, 2)
    wait_gather(3)
    wait_idx(dst_hbm, idb, sem_id, 3)
    issue_scatter(123, 3)
    wait_gather(0)
    wait_idx(dst_hbm, idb, sem_id, 0)
    issue_scatter(124, 0)
    wait_idx(src_hbm, isb, sem_is, 1)
    wait_scatter(1)
    wait_scatter(2)
    wait_scatter(3)
    wait_scatter(0)

    plsc.subcore_barrier()
    pltpu.sync_copy(acc.at[pl.ds(r0, ROWS_PT)],
                    out_hbm.at[pl.ds(cid * N_PAD + r0, ROWS_PT)])


_edge_pass = functools.partial(
    pl.kernel,
    mesh=_mesh,
    out_type=jax.ShapeDtypeStruct((NC * N_PAD, D), jnp.float32),
    scratch_types=(
        [pltpu.VMEM((CHUNK,), jnp.int32)] * 8
        + [pltpu.VMEM((CHUNK, D), jnp.float32)] * 4
        + [
            pltpu.VMEM_SHARED((N_PAD, D), jnp.float32),
            pltpu.SemaphoreType.DMA((NBUF,)),
            pltpu.SemaphoreType.DMA((NBUF,)),
            pltpu.SemaphoreType.DMA((NBUF,)),
            pltpu.SemaphoreType.DMA((NBUF,)),
        ]
    ),
)(_edge_body)


def _deg_body(dst_hbm, ones_hbm, zeros_hbm, out_hbm,
              id0, id1, id2, id3, id4, id5, id6, id7, ones_v, acc, sem_s, sem_i):
    cid = lax.axis_index("c")
    sid = lax.axis_index("s")
    r0 = sid * ROWS_PT
    pltpu.sync_copy(zeros_hbm.at[pl.ds(r0, ROWS_PT)], acc.at[pl.ds(r0, ROWS_PT)])
    pltpu.sync_copy(ones_hbm, ones_v)
    base = (cid * NS + sid) * EPT
    plsc.subcore_barrier()

    idb = [id0, id1, id2, id3, id4, id5, id6, id7]

    def load_idx(i, b8):
        pltpu.async_copy(dst_hbm.at[pl.ds(base + i * CHUNK, CHUNK)],
                         idb[b8], sem_i.at[b8])

    def issue_scatter(i, b8, bs):
        pltpu.make_async_copy(dst_hbm.at[pl.ds(base, CHUNK)],
                              idb[b8], sem_i.at[b8]).wait()
        pltpu.async_copy(ones_v, acc.at[idb[b8]], sem_s.at[bs], add=True)

    def wait_scatter(b):
        pltpu.make_async_copy(ones_v, acc.at[idb[0]], sem_s.at[b]).wait()

    for b in range(NBUF):
        load_idx(b, b)
    for b in range(NBUF):
        load_idx(b + 4, b + 4)
        issue_scatter(b, b, b)

    def steady(i, carry):
        for m in range(8):
            ci = i + m
            b8 = (4 + m) % 8
            wait_scatter(m % NBUF)
            load_idx(ci + 4, (b8 + 4) % 8)
            issue_scatter(ci, b8, m % NBUF)
        return carry

    lax.fori_loop(0, 14, lambda j, c: steady(8 * j + 4, c), 0)

    for ci in range(116, 125):
        wait_scatter(ci % NBUF)
        if ci + 4 <= 124:
            load_idx(ci + 4, (ci + 4) % 8)
        issue_scatter(ci, ci % 8, ci % NBUF)
    wait_scatter(1)
    wait_scatter(2)
    wait_scatter(3)
    wait_scatter(0)

    plsc.subcore_barrier()
    pltpu.sync_copy(acc.at[pl.ds(r0, ROWS_PT)],
                    out_hbm.at[pl.ds(cid * N_PAD + r0, ROWS_PT)])


_deg_pass = functools.partial(
    pl.kernel,
    mesh=_mesh,
    out_type=jax.ShapeDtypeStruct((NC * N_PAD, D), jnp.float32),
    scratch_types=(
        [pltpu.VMEM((CHUNK,), jnp.int32)] * 8
        + [
            pltpu.VMEM((CHUNK, D), jnp.float32),
            pltpu.VMEM_SHARED((N_PAD, D), jnp.float32),
            pltpu.SemaphoreType.DMA((NBUF,)),
            pltpu.SemaphoreType.DMA((8,)),
        ]
    ),
)(_deg_body)


def _dinv_of(deg_ref):
    deg = deg_ref[0:N_NODES] + deg_ref[N_PAD:N_PAD + N_NODES]
    degt = jnp.max(deg, axis=1, keepdims=True) + 1.0
    return lax.rsqrt(degt)


def _tc1_body(x_ref, w_ref, deg_ref, o_ref, dinv_ref):
    dinv = _dinv_of(deg_ref)
    h = jnp.dot(x_ref[...], w_ref[...], preferred_element_type=jnp.float32)
    o_ref[...] = h * dinv
    dinv_ref[...] = dinv


def _tc2_body(agg_ref, hhat_ref, dinv_ref, b_ref, w_ref, o_ref):
    dinv = dinv_ref[...]
    s = agg_ref[0:N_NODES] + agg_ref[N_PAD:N_PAD + N_NODES] + hhat_ref[...]
    a = jnp.maximum(s * dinv + b_ref[...], 0.0)
    h2 = jnp.dot(a, w_ref[...], preferred_element_type=jnp.float32)
    o_ref[...] = h2 * dinv


def _tc3_body(agg_ref, hhat_ref, dinv_ref, b_ref, wr_ref, br_ref, o_ref):
    dinv = dinv_ref[...]
    s = agg_ref[0:N_NODES] + agg_ref[N_PAD:N_PAD + N_NODES] + hhat_ref[...]
    a = jnp.maximum(s * dinv + b_ref[...], 0.0)
    z = jnp.dot(a, wr_ref[...], preferred_element_type=jnp.float32) + br_ref[...]
    o_ref[...] = jax.nn.sigmoid(z) * 0.8 + 0.1


_f32 = jnp.float32
_tc1 = pl.pallas_call(
    _tc1_body, out_shape=(jax.ShapeDtypeStruct((N_NODES, D), _f32),
                          jax.ShapeDtypeStruct((N_NODES, 1), _f32)))
_tc2 = pl.pallas_call(
    _tc2_body, out_shape=jax.ShapeDtypeStruct((N_NODES, D), _f32))
_tc3 = pl.pallas_call(
    _tc3_body, out_shape=jax.ShapeDtypeStruct((N_NODES, D), _f32))


@jax.jit
def kernel(x, edge_index, W1, b1, W2, b2, Wr, br):
    src = edge_index[0].astype(jnp.int32)
    dst = edge_index[1].astype(jnp.int32)
    zeros128 = jnp.asarray(_ZEROS128)
    ones128 = jnp.asarray(_ONES128)

    src_pad = jnp.concatenate([src, jnp.zeros((CHUNK,), jnp.int32)])

    degp = _deg_pass(dst, ones128, zeros128)
    hhat, dinv = _tc1(x, W1, degp)
    agg1 = _edge_pass(hhat, src_pad, dst, zeros128)
    hhat2 = _tc2(agg1, hhat, dinv, b1.reshape(1, D), W2)
    agg2 = _edge_pass(hhat2, src_pad, dst, zeros128)
    out = _tc3(agg2, hhat2, dinv, b2.reshape(1, D), Wr, br.reshape(1, D))
    return out


# speedup vs baseline: 1.0269x; 1.0146x over previous
import functools

import numpy as np

import jax
import jax.numpy as jnp
from jax import lax
from jax.experimental import pallas as pl
from jax.experimental.pallas import tpu as pltpu
from jax.experimental.pallas import tpu_sc as plsc

N_NODES = 10000
N_EDGES = 320000
D = 128
NC = 2
NS = 16
NW = NC * NS
EPT = N_EDGES // NW
CHUNK = 80
NCHUNK = EPT // CHUNK
NBUF = 4
N_PAD = 10240
ROWS_PT = N_PAD // NS

_mesh = plsc.VectorSubcoreMesh(core_axis_name="c", subcore_axis_name="s")

_ZEROS128 = np.zeros((ROWS_PT, D), np.float32)
_ONES128 = np.ones((CHUNK, D), np.float32)


def _edge_body(h_hbm, ei_hbm, zeros_hbm, out_hbm,
               is0, is1, is2, is3, id0, id1, id2, id3,
               rw0, rw1, rw2, rw3, acc, sem_g, sem_s, sem_is, sem_id):
    cid = lax.axis_index("c")
    sid = lax.axis_index("s")
    r0 = sid * ROWS_PT
    pltpu.sync_copy(zeros_hbm, acc.at[pl.ds(r0, ROWS_PT)])
    base = (cid * NS + sid) * EPT

    isb = [is0, is1, is2, is3]
    idb = [id0, id1, id2, id3]
    rows = [rw0, rw1, rw2, rw3]

    def load_idx_sync(doff, buf, i):
        pltpu.sync_copy(ei_hbm.at[pl.ds(doff + base + i * CHUNK, CHUNK)], buf)

    def load_idx_async(doff, buf, sem, i, b):
        pltpu.async_copy(ei_hbm.at[pl.ds(doff + base + i * CHUNK, CHUNK)],
                         buf[b], sem.at[b])

    def wait_idx(doff, buf, sem, b):
        pltpu.make_async_copy(ei_hbm.at[pl.ds(base, CHUNK)], buf[b], sem.at[b]).wait()

    def issue_gather(i, b):
        pltpu.async_copy(h_hbm.at[isb[b]], rows[b], sem_g.at[b])

    def wait_gather(b):
        pltpu.make_async_copy(h_hbm.at[isb[b]], rows[b], sem_g.at[b]).wait()

    def issue_scatter(i, b):
        pltpu.async_copy(rows[b], acc.at[idb[b]], sem_s.at[b], add=True)

    def wait_scatter(b):
        pltpu.make_async_copy(rows[b], acc.at[idb[b]], sem_s.at[b]).wait()

    for b in range(NBUF):
        load_idx_sync(0, isb[b], b)
    load_idx_sync(N_EDGES, idb[0], 0)
    load_idx_sync(N_EDGES, idb[1], 1)
    plsc.subcore_barrier()

    issue_gather(0, 0)
    issue_gather(1, 1)
    load_idx_async(N_EDGES, idb, sem_id, 2, 2)
    issue_gather(2, 2)
    wait_gather(0)
    load_idx_async(0, isb, sem_is, 4, 0)
    issue_scatter(0, 0)
    load_idx_async(N_EDGES, idb, sem_id, 3, 3)
    issue_gather(3, 3)
    wait_gather(1)
    load_idx_async(0, isb, sem_is, 5, 1)
    issue_scatter(1, 1)

    def steady(i, carry):
        for m in range(NBUF):
            ci = i + m
            p = (2 + m) % NBUF
            wait_scatter(m)
            load_idx_async(N_EDGES, idb, sem_id, ci + 2, m)
            wait_idx(0, isb, sem_is, m)
            issue_gather(ci + 2, m)
            wait_gather(p)
            load_idx_async(0, isb, sem_is, ci + 4, p)
            wait_idx(N_EDGES, idb, sem_id, p)
            issue_scatter(ci, p)
        return carry

    lax.fori_loop(0, 30, lambda j, c: steady(4 * j + 2, c), 0)

    wait_scatter(0)
    load_idx_async(N_EDGES, idb, sem_id, 124, 0)
    wait_idx(0, isb, sem_is, 0)
    issue_gather(124, 0)
    wait_gather(2)
    wait_idx(N_EDGES, idb, sem_id, 2)
    issue_scatter(122, 2)
    wait_gather(3)
    wait_idx(N_EDGES, idb, sem_id, 3)
    issue_scatter(123, 3)
    wait_gather(0)
    wait_idx(N_EDGES, idb, sem_id, 0)
    issue_scatter(124, 0)
    wait_idx(0, isb, sem_is, 1)
    wait_scatter(1)
    wait_scatter(2)
    wait_scatter(3)
    wait_scatter(0)

    plsc.subcore_barrier()
    pltpu.sync_copy(acc.at[pl.ds(r0, ROWS_PT)],
                    out_hbm.at[pl.ds(cid * N_PAD + r0, ROWS_PT)])


_edge_pass = functools.partial(
    pl.kernel,
    mesh=_mesh,
    out_type=jax.ShapeDtypeStruct((NC * N_PAD, D), jnp.float32),
    scratch_types=(
        [pltpu.VMEM((CHUNK,), jnp.int32)] * 8
        + [pltpu.VMEM((CHUNK, D), jnp.float32)] * 4
        + [
            pltpu.VMEM_SHARED((N_PAD, D), jnp.float32),
            pltpu.SemaphoreType.DMA((NBUF,)),
            pltpu.SemaphoreType.DMA((NBUF,)),
            pltpu.SemaphoreType.DMA((NBUF,)),
            pltpu.SemaphoreType.DMA((NBUF,)),
        ]
    ),
)(_edge_body)


def _deg_body(ei_hbm, ones_hbm, zeros_hbm, out_hbm,
              id0, id1, id2, id3, id4, id5, id6, id7, ones_v, acc, sem_s, sem_i):
    cid = lax.axis_index("c")
    sid = lax.axis_index("s")
    r0 = sid * ROWS_PT
    pltpu.sync_copy(zeros_hbm, acc.at[pl.ds(r0, ROWS_PT)])
    pltpu.sync_copy(ones_hbm, ones_v)
    base = (cid * NS + sid) * EPT
    plsc.subcore_barrier()

    idb = [id0, id1, id2, id3, id4, id5, id6, id7]

    def load_idx(i, b8):
        pltpu.async_copy(ei_hbm.at[pl.ds(N_EDGES + base + i * CHUNK, CHUNK)],
                         idb[b8], sem_i.at[b8])

    def issue_scatter(i, b8, bs):
        pltpu.make_async_copy(ei_hbm.at[pl.ds(base, CHUNK)],
                              idb[b8], sem_i.at[b8]).wait()
        pltpu.async_copy(ones_v, acc.at[idb[b8]], sem_s.at[bs], add=True)

    def wait_scatter(b):
        pltpu.make_async_copy(ones_v, acc.at[idb[0]], sem_s.at[b]).wait()

    for b in range(NBUF):
        load_idx(b, b)
    for b in range(NBUF):
        load_idx(b + 4, b + 4)
        issue_scatter(b, b, b)

    def steady(i, carry):
        for m in range(8):
            ci = i + m
            b8 = (4 + m) % 8
            wait_scatter(m % NBUF)
            load_idx(ci + 4, (b8 + 4) % 8)
            issue_scatter(ci, b8, m % NBUF)
        return carry

    lax.fori_loop(0, 14, lambda j, c: steady(8 * j + 4, c), 0)

    for ci in range(116, 125):
        wait_scatter(ci % NBUF)
        if ci + 4 <= 124:
            load_idx(ci + 4, (ci + 4) % 8)
        issue_scatter(ci, ci % 8, ci % NBUF)
    wait_scatter(1)
    wait_scatter(2)
    wait_scatter(3)
    wait_scatter(0)

    plsc.subcore_barrier()
    pltpu.sync_copy(acc.at[pl.ds(r0, ROWS_PT)],
                    out_hbm.at[pl.ds(cid * N_PAD + r0, ROWS_PT)])


_deg_pass = functools.partial(
    pl.kernel,
    mesh=_mesh,
    out_type=jax.ShapeDtypeStruct((NC * N_PAD, D), jnp.float32),
    scratch_types=(
        [pltpu.VMEM((CHUNK,), jnp.int32)] * 8
        + [
            pltpu.VMEM((CHUNK, D), jnp.float32),
            pltpu.VMEM_SHARED((N_PAD, D), jnp.float32),
            pltpu.SemaphoreType.DMA((NBUF,)),
            pltpu.SemaphoreType.DMA((8,)),
        ]
    ),
)(_deg_body)


def _dinv_of(deg_ref):
    deg = deg_ref[0:N_NODES] + deg_ref[N_PAD:N_PAD + N_NODES]
    degt = jnp.max(deg, axis=1, keepdims=True) + 1.0
    return lax.rsqrt(degt)


def _tc1_body(x_ref, w_ref, deg_ref, o_ref, dinv_ref):
    dinv = _dinv_of(deg_ref)
    h = jnp.dot(x_ref[...], w_ref[...], preferred_element_type=jnp.float32)
    o_ref[...] = h * dinv
    dinv_ref[...] = dinv


def _tc2_body(agg_ref, hhat_ref, dinv_ref, b_ref, w_ref, o_ref):
    dinv = dinv_ref[...]
    s = agg_ref[0:N_NODES] + agg_ref[N_PAD:N_PAD + N_NODES] + hhat_ref[...]
    a = jnp.maximum(s * dinv + b_ref[...], 0.0)
    h2 = jnp.dot(a, w_ref[...], preferred_element_type=jnp.float32)
    o_ref[...] = h2 * dinv


def _tc3_body(agg_ref, hhat_ref, dinv_ref, b_ref, wr_ref, br_ref, o_ref):
    dinv = dinv_ref[...]
    s = agg_ref[0:N_NODES] + agg_ref[N_PAD:N_PAD + N_NODES] + hhat_ref[...]
    a = jnp.maximum(s * dinv + b_ref[...], 0.0)
    z = jnp.dot(a, wr_ref[...], preferred_element_type=jnp.float32) + br_ref[...]
    o_ref[...] = jax.nn.sigmoid(z) * 0.8 + 0.1


_f32 = jnp.float32
_tc1 = pl.pallas_call(
    _tc1_body, out_shape=(jax.ShapeDtypeStruct((N_NODES, D), _f32),
                          jax.ShapeDtypeStruct((N_NODES, 1), _f32)))
_tc2 = pl.pallas_call(
    _tc2_body, out_shape=jax.ShapeDtypeStruct((N_NODES, D), _f32))
_tc3 = pl.pallas_call(
    _tc3_body, out_shape=jax.ShapeDtypeStruct((N_NODES, D), _f32))


@jax.jit
def kernel(x, edge_index, W1, b1, W2, b2, Wr, br):
    ei_flat = edge_index.astype(jnp.int32).reshape(2 * N_EDGES)
    zeros128 = jnp.asarray(_ZEROS128)
    ones128 = jnp.asarray(_ONES128)

    degp = _deg_pass(ei_flat, ones128, zeros128)
    hhat, dinv = _tc1(x, W1, degp)
    agg1 = _edge_pass(hhat, ei_flat, zeros128)
    hhat2 = _tc2(agg1, hhat, dinv, b1.reshape(1, D), W2)
    agg2 = _edge_pass(hhat2, ei_flat, zeros128)
    out = _tc3(agg2, hhat2, dinv, b2.reshape(1, D), Wr, br.reshape(1, D))
    return out
